# Initial kernel scaffold; baseline (speedup 1.0000x reference)
import jax
import jax.numpy as jnp
from jax.experimental import pallas as pl


def kernel(state, edge_index, W1, b1, W2, b2, W3, b3, lin1_W, lin1_b, lin2_W, lin2_b, lin3_W, lin3_b):
    raise NotImplementedError("write your pallas kernel here")


# speedup vs baseline: 2.9859x; 2.9859x over previous
import functools

import jax
import jax.numpy as jnp
from jax import lax
from jax.experimental import pallas as pl
from jax.experimental.pallas import tpu as pltpu
from jax.experimental.pallas import tpu_sc as plsc
import numpy as np

N = 79000
C = 64
E = 1264000
ACT = 79
BATCH = N // ACT

NP = 79872
NH = NP // 2
NSUB = 16
KL = 128
KC = 16
K = KL * KC
CHUNKS = 40
PER_TILE = CHUNKS * K
EP = PER_TILE * NSUB
ZST = (NH + KL) // NSUB
OST = NH // NSUB
NBLK = 768
NGRID = NP // NBLK
HBLK = 40 * ACT
HGRID = BATCH // 40

_POS_INDICES = [120, 124, 128, 132, 136, 140, 144, 148, 152, 237, 241, 245,
                249, 253, 257, 261, 265, 269, 354, 358, 362, 366, 370, 374,
                378, 382, 386, 471, 475, 479, 483, 487, 491, 495, 499, 503,
                588, 592, 596, 600, 604, 608, 612, 616, 620, 705, 709, 713,
                717, 721, 725, 729, 733, 737, 822, 826, 830, 834, 838, 842,
                846, 850, 854, 48, 53, 60, 67, 73, 157, 352, 388, 583, 586,
                817, 901, 906, 913, 920, 926]


def _positions():
    width, height = 39, 25
    pf = np.zeros((ACT, 2), dtype=np.float32)
    for i, p in enumerate(_POS_INDICES):
        pf[i, 0] = (p % width) / (width - 1)
        pf[i, 1] = (p // width) / (height - 1)
    return jnp.asarray(pf)


def _sc_deg_body(colsc, out, idx_s, ones_b, zbig, acc):
    cid = lax.axis_index("c")
    sid = lax.axis_index("s")

    @pl.loop(0, KL)
    def _fill(i):
        ones_b[i, :] = jnp.full((16,), 1.0, jnp.float32)

    @pl.loop(0, ZST)
    def _fz(i):
        zbig[i, :] = jnp.zeros((16,), jnp.float32)

    pltpu.sync_copy(zbig, acc.at[pl.ds(sid * ZST, ZST), :])
    plsc.subcore_barrier()

    @pl.loop(0, CHUNKS)
    def _chunk(ch):
        pltpu.sync_copy(colsc.at[cid, sid, ch], idx_s)
        for j in range(KC):
            pltpu.sync_copy(ones_b, acc.at[idx_s.at[j]], add=True)

    plsc.subcore_barrier()
    pltpu.sync_copy(acc.at[pl.ds(sid * OST, OST), :],
                    out.at[pl.ds(cid * NH + sid * OST, OST)])


def _sc_edge_body(hs_flat, rowsq, colsc, out, idx_g, idx_s, rows, zbig, acc,
                  sem):
    cid = lax.axis_index("c")
    sid = lax.axis_index("s")

    @pl.loop(0, ZST)
    def _fz(i):
        zbig[i, :] = jnp.zeros((16,), jnp.float32)

    pltpu.sync_copy(zbig, acc.at[pl.ds(sid * ZST, ZST), :])
    plsc.subcore_barrier()

    @pl.loop(0, CHUNKS)
    def _chunk(ch):
        pltpu.sync_copy(rowsq.at[sid, ch], idx_g)
        pltpu.sync_copy(colsc.at[cid, sid, ch], idx_s)
        copies = []
        for j in range(KC):
            copies.append(pltpu.async_copy(
                hs_flat.at[idx_g.at[j]],
                rows.at[pl.ds(j * KL, KL), :], sem))
        for cp in copies:
            cp.wait()
        for j in range(KC):
            pltpu.sync_copy(rows.at[pl.ds(j * KL, KL), :],
                            acc.at[idx_s.at[j]], add=True)

    plsc.subcore_barrier()
    pltpu.sync_copy(acc.at[pl.ds(sid * OST, OST), :],
                    out.at[pl.ds(cid * NH + sid * OST, OST)])


@functools.lru_cache(maxsize=1)
def _sc_kernels():
    mesh = plsc.VectorSubcoreMesh(core_axis_name="c", subcore_axis_name="s")
    params = pltpu.CompilerParams(use_tc_tiling_on_sc=False)
    sc_deg = functools.partial(
        pl.kernel,
        out_type=jax.ShapeDtypeStruct((NP, 16), jnp.float32),
        mesh=mesh,
        scratch_types=[
            pltpu.VMEM((KC, KL), jnp.int32),
            pltpu.VMEM((KL, 16), jnp.float32),
            pltpu.VMEM((ZST, 16), jnp.float32),
            pltpu.VMEM_SHARED((NH + KL, 16), jnp.float32),
        ],
        compiler_params=params,
    )(_sc_deg_body)
    sc_edge = functools.partial(
        pl.kernel,
        out_type=jax.ShapeDtypeStruct((NP, 16), jnp.float32),
        mesh=mesh,
        scratch_types=[
            pltpu.VMEM((KC, KL), jnp.int32),
            pltpu.VMEM((KC, KL), jnp.int32),
            pltpu.VMEM((K, 16), jnp.float32),
            pltpu.VMEM((ZST, 16), jnp.float32),
            pltpu.VMEM_SHARED((NH + KL, 16), jnp.float32),
            pltpu.SemaphoreType.DMA,
        ],
        compiler_params=params,
    )(_sc_edge_body)
    return sc_deg, sc_edge


def _t1_body(deg_ref, state_ref, w_ref, dis_ref, hs_ref):
    deg = deg_ref[:, 0:1]
    dis = lax.rsqrt(deg + 1.0)
    dis_ref[...] = dis
    hs_ref[...] = jnp.dot(state_ref[...], w_ref[...],
                          preferred_element_type=jnp.float32) * dis


def _t1(deg, state_p, w1):
    return pl.pallas_call(
        _t1_body,
        grid=(NGRID,),
        in_specs=[
            pl.BlockSpec((NBLK, 16), lambda i: (i, 0)),
            pl.BlockSpec((NBLK, C), lambda i: (i, 0)),
            pl.BlockSpec((C, C), lambda i: (0, 0)),
        ],
        out_specs=[
            pl.BlockSpec((NBLK, 1), lambda i: (i, 0)),
            pl.BlockSpec((NBLK, C), lambda i: (i, 0)),
        ],
        out_shape=[
            jax.ShapeDtypeStruct((NP, 1), jnp.float32),
            jax.ShapeDtypeStruct((NP, C), jnp.float32),
        ],
    )(deg, state_p, w1)


def _tc_body(a0, a1, a2, a3, hs_ref, dis_ref, b_ref, w_ref, x_ref, hsn_ref):
    dis = dis_ref[...]
    accs = [a0, a1, a2, a3]
    for q in range(4):
        aq = accs[q][...] + hs_ref[:, q * 16:(q + 1) * 16]
        pre = dis * aq + b_ref[:, q * 16:(q + 1) * 16]
        x_ref[:, q * 16:(q + 1) * 16] = jnp.maximum(pre, 0.0)
    x = x_ref[...]
    hsn_ref[...] = jnp.dot(x, w_ref[...],
                           preferred_element_type=jnp.float32) * dis


def _tc_combine(accs, hs, dis, b_row, w_next):
    qspec = pl.BlockSpec((NBLK, 16), lambda i: (i, 0))
    return pl.pallas_call(
        _tc_body,
        grid=(NGRID,),
        in_specs=[
            qspec, qspec, qspec, qspec,
            pl.BlockSpec((NBLK, C), lambda i: (i, 0)),
            pl.BlockSpec((NBLK, 1), lambda i: (i, 0)),
            pl.BlockSpec((1, C), lambda i: (0, 0)),
            pl.BlockSpec((C, C), lambda i: (0, 0)),
        ],
        out_specs=[
            pl.BlockSpec((NBLK, C), lambda i: (i, 0)),
            pl.BlockSpec((NBLK, C), lambda i: (i, 0)),
        ],
        out_shape=[
            jax.ShapeDtypeStruct((NP, C), jnp.float32),
            jax.ShapeDtypeStruct((NP, C), jnp.float32),
        ],
    )(*accs, hs, dis, b_row, w_next)


def _tc_last_body(a0, a1, a2, a3, hs_ref, dis_ref, b_ref, x_ref):
    dis = dis_ref[...]
    accs = [a0, a1, a2, a3]
    for q in range(4):
        aq = accs[q][...] + hs_ref[:, q * 16:(q + 1) * 16]
        pre = dis * aq + b_ref[:, q * 16:(q + 1) * 16]
        x_ref[:, q * 16:(q + 1) * 16] = jnp.maximum(pre, 0.0)


def _tc_last(accs, hs, dis, b_row):
    qspec = pl.BlockSpec((NBLK, 16), lambda i: (i, 0))
    return pl.pallas_call(
        _tc_last_body,
        grid=(NGRID,),
        in_specs=[
            qspec, qspec, qspec, qspec,
            pl.BlockSpec((NBLK, C), lambda i: (i, 0)),
            pl.BlockSpec((NBLK, 1), lambda i: (i, 0)),
            pl.BlockSpec((1, C), lambda i: (0, 0)),
        ],
        out_specs=pl.BlockSpec((NBLK, C), lambda i: (i, 0)),
        out_shape=jax.ShapeDtypeStruct((NP, C), jnp.float32),
    )(*accs, hs, dis, b_row)


def _head_body(x1, x2, x3, x4, x5, st, w1s_ref, ws_ref, wt_ref, posb_ref,
               w2_ref, b2_ref, w3_ref, b3_ref, act_ref, reg_ref):
    f32 = jnp.float32
    y = posb_ref[...]
    xs = [x1, x2, x3, x4, x5]
    for l in range(5):
        y = y + jnp.dot(xs[l][...], w1s_ref[l], preferred_element_type=f32)
    y = y + jnp.dot(st[...], ws_ref[...], preferred_element_type=f32)
    nb = HBLK // ACT
    m_bn = (lax.broadcasted_iota(jnp.int32, (nb, HBLK), 1) // ACT ==
            lax.broadcasted_iota(jnp.int32, (nb, HBLK), 0)).astype(f32)
    m_nb = (lax.broadcasted_iota(jnp.int32, (HBLK, nb), 0) // ACT ==
            lax.broadcasted_iota(jnp.int32, (HBLK, nb), 1)).astype(f32)
    stc = st[:, 1:2]
    tot = jnp.dot(m_bn, stc, preferred_element_type=f32)
    tot_pn = jnp.dot(m_nb, tot, preferred_element_type=f32)
    y = y + tot_pn * wt_ref[...]
    y = jnp.where(y > 0, y, 0.01 * y)
    z = jnp.dot(y, w2_ref[...], preferred_element_type=f32) + b2_ref[...]
    z = jnp.where(z > 0, z, 0.01 * z)
    u = jnp.dot(z, w3_ref[...], preferred_element_type=f32) + b3_ref[...]
    conc = jnp.maximum(u, 0.0) + jnp.log(1.0 + jnp.exp(-jnp.abs(u)))
    den = jnp.dot(m_bn, conc, preferred_element_type=f32)
    den_pn = jnp.dot(m_nb, den, preferred_element_type=f32)
    act_ref[...] = conc / (den_pn + 1e-20)
    s = jnp.sum(jnp.abs(conc), keepdims=True)
    i = pl.program_id(0)

    @pl.when(i == 0)
    def _init():
        reg_ref[...] = s

    @pl.when(i != 0)
    def _acc():
        reg_ref[...] = reg_ref[...] + s


def _head(x1, x2, x3, x4, x5, state_p, w1_stack, ws, wt, posb, w2, b2, w3, b3):
    xspec = pl.BlockSpec((HBLK, C), lambda i: (i, 0))
    return pl.pallas_call(
        _head_body,
        grid=(HGRID,),
        in_specs=[
            xspec, xspec, xspec, xspec, xspec, xspec,
            pl.BlockSpec((5, C, 32), lambda i: (0, 0, 0)),
            pl.BlockSpec((C, 32), lambda i: (0, 0)),
            pl.BlockSpec((1, 32), lambda i: (0, 0)),
            pl.BlockSpec((HBLK, 32), lambda i: (0, 0)),
            pl.BlockSpec((32, 32), lambda i: (0, 0)),
            pl.BlockSpec((1, 32), lambda i: (0, 0)),
            pl.BlockSpec((32, 1), lambda i: (0, 0)),
            pl.BlockSpec((1, 1), lambda i: (0, 0)),
        ],
        out_specs=[
            pl.BlockSpec((HBLK, 1), lambda i: (i, 0)),
            pl.BlockSpec((1, 1), lambda i: (0, 0)),
        ],
        out_shape=[
            jax.ShapeDtypeStruct((N, 1), jnp.float32),
            jax.ShapeDtypeStruct((1, 1), jnp.float32),
        ],
    )(x1, x2, x3, x4, x5, state_p, w1_stack, ws, wt, posb, w2, b2, w3, b3)


def kernel(state, edge_index, W1, b1, W2, b2, W3, b3,
           lin1_W, lin1_b, lin2_W, lin2_b, lin3_W, lin3_b):
    i32 = jnp.int32
    f32 = jnp.float32

    row = edge_index[0]
    col = edge_index[1]
    row_p = jnp.concatenate([row, jnp.zeros((EP - E,), i32)])
    col_p = jnp.concatenate([col, jnp.full((EP - E,), N, i32)])
    rows4 = (row_p[None, :] * 4 + jnp.arange(4, dtype=i32)[:, None]).reshape(
        4, NSUB, CHUNKS, KC, KL)
    local0 = jnp.where((col_p >= 0) & (col_p < NH), col_p, NH)
    local1c = col_p - NH
    local1 = jnp.where((local1c >= 0) & (local1c < NH), local1c, NH)
    colsc = jnp.stack([local0, local1]).reshape(2, NSUB, CHUNKS, KC, KL)

    state_p = jnp.zeros((NP, C), f32).at[:N].set(state)

    sc_deg, sc_edge = _sc_kernels()

    def edge_pass(hs):
        hs_flat = hs.reshape(4 * NP, 16)
        return [sc_edge(hs_flat, rows4[q], colsc) for q in range(4)]

    deg = sc_deg(colsc)
    dis, hs1 = _t1(deg, state_p, W1)

    b1r = b1.reshape(1, C)
    b2r = b2.reshape(1, C)
    b3r = b3.reshape(1, C)

    x1, hs2 = _tc_combine(edge_pass(hs1), hs1, dis, b1r, W2)
    x2, hs3 = _tc_combine(edge_pass(hs2), hs2, dis, b2r, W3)
    x3, hs4 = _tc_combine(edge_pass(hs3), hs3, dis, b3r, W3)
    x4, hs5 = _tc_combine(edge_pass(hs4), hs4, dis, b3r, W3)
    x5 = _tc_last(edge_pass(hs5), hs5, dis, b3r)

    w1_stack = lin1_W[:5 * C].reshape(5, C, 32)
    ws = lin1_W[5 * C:6 * C]
    wt = lin1_W[6 * C].reshape(1, 32)
    pos = _positions()
    pos_lin = pos @ lin1_W[6 * C + 1:] + lin1_b[None, :]
    posb = jnp.tile(pos_lin, (HBLK // ACT, 1))
    b2h = lin2_b.reshape(1, 32)
    b3h = lin3_b.reshape(1, 1)

    act_col, reg = _head(x1, x2, x3, x4, x5, state_p, w1_stack, ws, wt,
                         posb, lin2_W, b2h, lin3_W, b3h)

    action = act_col.reshape(BATCH, ACT)
    regularize = reg[0, 0] / jnp.float32(N)
    return (action, regularize)


# speedup vs baseline: 2.9895x; 1.0012x over previous
import functools

import jax
import jax.numpy as jnp
from jax import lax
from jax.experimental import pallas as pl
from jax.experimental.pallas import tpu as pltpu
from jax.experimental.pallas import tpu_sc as plsc
import numpy as np

N = 79000
C = 64
E = 1264000
ACT = 79
BATCH = N // ACT

NP = 79872
NH = NP // 2
NSUB = 16
KL = 128
KC = 16
K = KL * KC
CHUNKS = 40
PER_TILE = CHUNKS * K
EP = PER_TILE * NSUB
ZST = (NH + KL) // NSUB
OST = NH // NSUB
NBLK = 768
NGRID = NP // NBLK
HBLK = 40 * ACT
HGRID = BATCH // 40

_POS_INDICES = [120, 124, 128, 132, 136, 140, 144, 148, 152, 237, 241, 245,
                249, 253, 257, 261, 265, 269, 354, 358, 362, 366, 370, 374,
                378, 382, 386, 471, 475, 479, 483, 487, 491, 495, 499, 503,
                588, 592, 596, 600, 604, 608, 612, 616, 620, 705, 709, 713,
                717, 721, 725, 729, 733, 737, 822, 826, 830, 834, 838, 842,
                846, 850, 854, 48, 53, 60, 67, 73, 157, 352, 388, 583, 586,
                817, 901, 906, 913, 920, 926]


def _positions():
    width, height = 39, 25
    pf = np.zeros((ACT, 2), dtype=np.float32)
    for i, p in enumerate(_POS_INDICES):
        pf[i, 0] = (p % width) / (width - 1)
        pf[i, 1] = (p // width) / (height - 1)
    return jnp.asarray(pf)


def _sc_deg_body(colsc, out, idx_s, ones_b, zbig, acc):
    cid = lax.axis_index("c")
    sid = lax.axis_index("s")

    @pl.loop(0, KL)
    def _fill(i):
        ones_b[i, :] = jnp.full((16,), 1.0, jnp.float32)

    @pl.loop(0, ZST)
    def _fz(i):
        zbig[i, :] = jnp.zeros((16,), jnp.float32)

    pltpu.sync_copy(zbig, acc.at[pl.ds(sid * ZST, ZST), :])
    plsc.subcore_barrier()

    @pl.loop(0, CHUNKS)
    def _chunk(ch):
        pltpu.sync_copy(colsc.at[cid, sid, ch], idx_s)
        for j in range(KC):
            pltpu.sync_copy(ones_b, acc.at[idx_s.at[j]], add=True)

    plsc.subcore_barrier()
    pltpu.sync_copy(acc.at[pl.ds(sid * OST, OST), :],
                    out.at[pl.ds(cid * NH + sid * OST, OST)])


def _sc_edge_body(hs_flat, rowsq, colsc, out, idx_g, idx_s, rows, zbig, acc,
                  sem, sem2):
    cid = lax.axis_index("c")
    sid = lax.axis_index("s")

    @pl.loop(0, ZST)
    def _fz(i):
        zbig[i, :] = jnp.zeros((16,), jnp.float32)

    pltpu.sync_copy(zbig, acc.at[pl.ds(sid * ZST, ZST), :])
    plsc.subcore_barrier()

    @pl.loop(0, CHUNKS)
    def _chunk(ch):
        pltpu.sync_copy(rowsq.at[sid, ch], idx_g)
        pltpu.sync_copy(colsc.at[cid, sid, ch], idx_s)
        gathers = []
        for j in range(KC):
            gathers.append(pltpu.async_copy(
                hs_flat.at[idx_g.at[j]],
                rows.at[pl.ds(j * KL, KL), :], sem))
        for cp in gathers:
            cp.wait()
        scatters = []
        for j in range(KC):
            scatters.append(pltpu.async_copy(
                rows.at[pl.ds(j * KL, KL), :],
                acc.at[idx_s.at[j]], sem2, add=True))
        for cp in scatters:
            cp.wait()

    plsc.subcore_barrier()
    pltpu.sync_copy(acc.at[pl.ds(sid * OST, OST), :],
                    out.at[pl.ds(cid * NH + sid * OST, OST)])


@functools.lru_cache(maxsize=1)
def _sc_kernels():
    mesh = plsc.VectorSubcoreMesh(core_axis_name="c", subcore_axis_name="s")
    params = pltpu.CompilerParams(use_tc_tiling_on_sc=False)
    sc_deg = functools.partial(
        pl.kernel,
        out_type=jax.ShapeDtypeStruct((NP, 16), jnp.float32),
        mesh=mesh,
        scratch_types=[
            pltpu.VMEM((KC, KL), jnp.int32),
            pltpu.VMEM((KL, 16), jnp.float32),
            pltpu.VMEM((ZST, 16), jnp.float32),
            pltpu.VMEM_SHARED((NH + KL, 16), jnp.float32),
        ],
        compiler_params=params,
    )(_sc_deg_body)
    sc_edge = functools.partial(
        pl.kernel,
        out_type=jax.ShapeDtypeStruct((NP, 16), jnp.float32),
        mesh=mesh,
        scratch_types=[
            pltpu.VMEM((KC, KL), jnp.int32),
            pltpu.VMEM((KC, KL), jnp.int32),
            pltpu.VMEM((K, 16), jnp.float32),
            pltpu.VMEM((ZST, 16), jnp.float32),
            pltpu.VMEM_SHARED((NH + KL, 16), jnp.float32),
            pltpu.SemaphoreType.DMA,
            pltpu.SemaphoreType.DMA,
        ],
        compiler_params=params,
    )(_sc_edge_body)
    return sc_deg, sc_edge


def _t1_body(deg_ref, state_ref, w_ref, dis_ref, hs_ref):
    deg = deg_ref[:, 0:1]
    dis = lax.rsqrt(deg + 1.0)
    dis_ref[...] = dis
    hs_ref[...] = jnp.dot(state_ref[...], w_ref[...],
                          preferred_element_type=jnp.float32) * dis


def _t1(deg, state_p, w1):
    return pl.pallas_call(
        _t1_body,
        grid=(NGRID,),
        in_specs=[
            pl.BlockSpec((NBLK, 16), lambda i: (i, 0)),
            pl.BlockSpec((NBLK, C), lambda i: (i, 0)),
            pl.BlockSpec((C, C), lambda i: (0, 0)),
        ],
        out_specs=[
            pl.BlockSpec((NBLK, 1), lambda i: (i, 0)),
            pl.BlockSpec((NBLK, C), lambda i: (i, 0)),
        ],
        out_shape=[
            jax.ShapeDtypeStruct((NP, 1), jnp.float32),
            jax.ShapeDtypeStruct((NP, C), jnp.float32),
        ],
    )(deg, state_p, w1)


def _tc_body(a0, a1, a2, a3, hs_ref, dis_ref, b_ref, w_ref, x_ref, hsn_ref):
    dis = dis_ref[...]
    accs = [a0, a1, a2, a3]
    for q in range(4):
        aq = accs[q][...] + hs_ref[:, q * 16:(q + 1) * 16]
        pre = dis * aq + b_ref[:, q * 16:(q + 1) * 16]
        x_ref[:, q * 16:(q + 1) * 16] = jnp.maximum(pre, 0.0)
    x = x_ref[...]
    hsn_ref[...] = jnp.dot(x, w_ref[...],
                           preferred_element_type=jnp.float32) * dis


def _tc_combine(accs, hs, dis, b_row, w_next):
    qspec = pl.BlockSpec((NBLK, 16), lambda i: (i, 0))
    return pl.pallas_call(
        _tc_body,
        grid=(NGRID,),
        in_specs=[
            qspec, qspec, qspec, qspec,
            pl.BlockSpec((NBLK, C), lambda i: (i, 0)),
            pl.BlockSpec((NBLK, 1), lambda i: (i, 0)),
            pl.BlockSpec((1, C), lambda i: (0, 0)),
            pl.BlockSpec((C, C), lambda i: (0, 0)),
        ],
        out_specs=[
            pl.BlockSpec((NBLK, C), lambda i: (i, 0)),
            pl.BlockSpec((NBLK, C), lambda i: (i, 0)),
        ],
        out_shape=[
            jax.ShapeDtypeStruct((NP, C), jnp.float32),
            jax.ShapeDtypeStruct((NP, C), jnp.float32),
        ],
    )(*accs, hs, dis, b_row, w_next)


def _tc_last_body(a0, a1, a2, a3, hs_ref, dis_ref, b_ref, x_ref):
    dis = dis_ref[...]
    accs = [a0, a1, a2, a3]
    for q in range(4):
        aq = accs[q][...] + hs_ref[:, q * 16:(q + 1) * 16]
        pre = dis * aq + b_ref[:, q * 16:(q + 1) * 16]
        x_ref[:, q * 16:(q + 1) * 16] = jnp.maximum(pre, 0.0)


def _tc_last(accs, hs, dis, b_row):
    qspec = pl.BlockSpec((NBLK, 16), lambda i: (i, 0))
    return pl.pallas_call(
        _tc_last_body,
        grid=(NGRID,),
        in_specs=[
            qspec, qspec, qspec, qspec,
            pl.BlockSpec((NBLK, C), lambda i: (i, 0)),
            pl.BlockSpec((NBLK, 1), lambda i: (i, 0)),
            pl.BlockSpec((1, C), lambda i: (0, 0)),
        ],
        out_specs=pl.BlockSpec((NBLK, C), lambda i: (i, 0)),
        out_shape=jax.ShapeDtypeStruct((NP, C), jnp.float32),
    )(*accs, hs, dis, b_row)


def _head_body(x1, x2, x3, x4, x5, st, w1s_ref, ws_ref, wt_ref, posb_ref,
               w2_ref, b2_ref, w3_ref, b3_ref, act_ref, reg_ref):
    f32 = jnp.float32
    y = posb_ref[...]
    xs = [x1, x2, x3, x4, x5]
    for l in range(5):
        y = y + jnp.dot(xs[l][...], w1s_ref[l], preferred_element_type=f32)
    y = y + jnp.dot(st[...], ws_ref[...], preferred_element_type=f32)
    nb = HBLK // ACT
    m_bn = (lax.broadcasted_iota(jnp.int32, (nb, HBLK), 1) // ACT ==
            lax.broadcasted_iota(jnp.int32, (nb, HBLK), 0)).astype(f32)
    m_nb = (lax.broadcasted_iota(jnp.int32, (HBLK, nb), 0) // ACT ==
            lax.broadcasted_iota(jnp.int32, (HBLK, nb), 1)).astype(f32)
    stc = st[:, 1:2]
    tot = jnp.dot(m_bn, stc, preferred_element_type=f32)
    tot_pn = jnp.dot(m_nb, tot, preferred_element_type=f32)
    y = y + tot_pn * wt_ref[...]
    y = jnp.where(y > 0, y, 0.01 * y)
    z = jnp.dot(y, w2_ref[...], preferred_element_type=f32) + b2_ref[...]
    z = jnp.where(z > 0, z, 0.01 * z)
    u = jnp.dot(z, w3_ref[...], preferred_element_type=f32) + b3_ref[...]
    conc = jnp.maximum(u, 0.0) + jnp.log(1.0 + jnp.exp(-jnp.abs(u)))
    den = jnp.dot(m_bn, conc, preferred_element_type=f32)
    den_pn = jnp.dot(m_nb, den, preferred_element_type=f32)
    act_ref[...] = conc / (den_pn + 1e-20)
    s = jnp.sum(jnp.abs(conc), keepdims=True)
    i = pl.program_id(0)

    @pl.when(i == 0)
    def _init():
        reg_ref[...] = s

    @pl.when(i != 0)
    def _acc():
        reg_ref[...] = reg_ref[...] + s


def _head(x1, x2, x3, x4, x5, state_p, w1_stack, ws, wt, posb, w2, b2, w3, b3):
    xspec = pl.BlockSpec((HBLK, C), lambda i: (i, 0))
    return pl.pallas_call(
        _head_body,
        grid=(HGRID,),
        in_specs=[
            xspec, xspec, xspec, xspec, xspec, xspec,
            pl.BlockSpec((5, C, 32), lambda i: (0, 0, 0)),
            pl.BlockSpec((C, 32), lambda i: (0, 0)),
            pl.BlockSpec((1, 32), lambda i: (0, 0)),
            pl.BlockSpec((HBLK, 32), lambda i: (0, 0)),
            pl.BlockSpec((32, 32), lambda i: (0, 0)),
            pl.BlockSpec((1, 32), lambda i: (0, 0)),
            pl.BlockSpec((32, 1), lambda i: (0, 0)),
            pl.BlockSpec((1, 1), lambda i: (0, 0)),
        ],
        out_specs=[
            pl.BlockSpec((HBLK, 1), lambda i: (i, 0)),
            pl.BlockSpec((1, 1), lambda i: (0, 0)),
        ],
        out_shape=[
            jax.ShapeDtypeStruct((N, 1), jnp.float32),
            jax.ShapeDtypeStruct((1, 1), jnp.float32),
        ],
    )(x1, x2, x3, x4, x5, state_p, w1_stack, ws, wt, posb, w2, b2, w3, b3)


def kernel(state, edge_index, W1, b1, W2, b2, W3, b3,
           lin1_W, lin1_b, lin2_W, lin2_b, lin3_W, lin3_b):
    i32 = jnp.int32
    f32 = jnp.float32

    row = edge_index[0]
    col = edge_index[1]
    row_p = jnp.concatenate([row, jnp.zeros((EP - E,), i32)])
    col_p = jnp.concatenate([col, jnp.full((EP - E,), N, i32)])
    rows4 = (row_p[None, :] * 4 + jnp.arange(4, dtype=i32)[:, None]).reshape(
        4, NSUB, CHUNKS, KC, KL)
    local0 = jnp.where((col_p >= 0) & (col_p < NH), col_p, NH)
    local1c = col_p - NH
    local1 = jnp.where((local1c >= 0) & (local1c < NH), local1c, NH)
    colsc = jnp.stack([local0, local1]).reshape(2, NSUB, CHUNKS, KC, KL)

    state_p = jnp.zeros((NP, C), f32).at[:N].set(state)

    sc_deg, sc_edge = _sc_kernels()

    def edge_pass(hs):
        hs_flat = hs.reshape(4 * NP, 16)
        return [sc_edge(hs_flat, rows4[q], colsc) for q in range(4)]

    deg = sc_deg(colsc)
    dis, hs1 = _t1(deg, state_p, W1)

    b1r = b1.reshape(1, C)
    b2r = b2.reshape(1, C)
    b3r = b3.reshape(1, C)

    x1, hs2 = _tc_combine(edge_pass(hs1), hs1, dis, b1r, W2)
    x2, hs3 = _tc_combine(edge_pass(hs2), hs2, dis, b2r, W3)
    x3, hs4 = _tc_combine(edge_pass(hs3), hs3, dis, b3r, W3)
    x4, hs5 = _tc_combine(edge_pass(hs4), hs4, dis, b3r, W3)
    x5 = _tc_last(edge_pass(hs5), hs5, dis, b3r)

    w1_stack = lin1_W[:5 * C].reshape(5, C, 32)
    ws = lin1_W[5 * C:6 * C]
    wt = lin1_W[6 * C].reshape(1, 32)
    pos = _positions()
    pos_lin = pos @ lin1_W[6 * C + 1:] + lin1_b[None, :]
    posb = jnp.tile(pos_lin, (HBLK // ACT, 1))
    b2h = lin2_b.reshape(1, 32)
    b3h = lin3_b.reshape(1, 1)

    act_col, reg = _head(x1, x2, x3, x4, x5, state_p, w1_stack, ws, wt,
                         posb, lin2_W, b2h, lin3_W, b3h)

    action = act_col.reshape(BATCH, ACT)
    regularize = reg[0, 0] / jnp.float32(N)
    return (action, regularize)


# speedup vs baseline: 3.0440x; 1.0183x over previous
import functools

import jax
import jax.numpy as jnp
from jax import lax
from jax.experimental import pallas as pl
from jax.experimental.pallas import tpu as pltpu
from jax.experimental.pallas import tpu_sc as plsc
import numpy as np

N = 79000
C = 64
E = 1264000
ACT = 79
BATCH = N // ACT

NP = 79872
NH = NP // 2
NSUB = 16
KL = 128
KC = 16
K = KL * KC
CHUNKS = 40
PER_TILE = CHUNKS * K
EP = PER_TILE * NSUB
ZST = (NH + KL) // NSUB
OST = NH // NSUB
NBLK = 768
NGRID = NP // NBLK
HBLK = 40 * ACT
HGRID = BATCH // 40

_POS_INDICES = [120, 124, 128, 132, 136, 140, 144, 148, 152, 237, 241, 245,
                249, 253, 257, 261, 265, 269, 354, 358, 362, 366, 370, 374,
                378, 382, 386, 471, 475, 479, 483, 487, 491, 495, 499, 503,
                588, 592, 596, 600, 604, 608, 612, 616, 620, 705, 709, 713,
                717, 721, 725, 729, 733, 737, 822, 826, 830, 834, 838, 842,
                846, 850, 854, 48, 53, 60, 67, 73, 157, 352, 388, 583, 586,
                817, 901, 906, 913, 920, 926]


def _positions():
    width, height = 39, 25
    pf = np.zeros((ACT, 2), dtype=np.float32)
    for i, p in enumerate(_POS_INDICES):
        pf[i, 0] = (p % width) / (width - 1)
        pf[i, 1] = (p // width) / (height - 1)
    return jnp.asarray(pf)


def _sc_deg_body(colsc, out, idx_s, ones_b, zbig, acc):
    cid = lax.axis_index("c")
    sid = lax.axis_index("s")

    @pl.loop(0, K)
    def _fill(i):
        ones_b[i, :] = jnp.full((16,), 1.0, jnp.float32)

    @pl.loop(0, ZST)
    def _fz(i):
        zbig[i, :] = jnp.zeros((16,), jnp.float32)

    pltpu.sync_copy(zbig, acc.at[pl.ds(sid * ZST, ZST), :])
    plsc.subcore_barrier()

    @pl.loop(0, CHUNKS)
    def _chunk(ch):
        pltpu.sync_copy(colsc.at[cid, sid, ch], idx_s)
        pltpu.sync_copy(ones_b, acc.at[idx_s], add=True)

    plsc.subcore_barrier()
    pltpu.sync_copy(acc.at[pl.ds(sid * OST, OST), :],
                    out.at[pl.ds(cid * NH + sid * OST, OST)])


def _sc_edge_body(hs_flat, rowsq, colsc, out, idx_g, idx_s, rows, zbig, acc,
                  sem, sem2):
    cid = lax.axis_index("c")
    sid = lax.axis_index("s")

    @pl.loop(0, ZST)
    def _fz(i):
        zbig[i, :] = jnp.zeros((16,), jnp.float32)

    pltpu.sync_copy(zbig, acc.at[pl.ds(sid * ZST, ZST), :])
    plsc.subcore_barrier()

    @pl.loop(0, CHUNKS)
    def _chunk(ch):
        pltpu.sync_copy(rowsq.at[sid, ch], idx_g)
        pltpu.sync_copy(colsc.at[cid, sid, ch], idx_s)
        pltpu.async_copy(hs_flat.at[idx_g], rows, sem).wait()
        pltpu.async_copy(rows, acc.at[idx_s], sem2, add=True).wait()

    plsc.subcore_barrier()
    pltpu.sync_copy(acc.at[pl.ds(sid * OST, OST), :],
                    out.at[pl.ds(cid * NH + sid * OST, OST)])


@functools.lru_cache(maxsize=1)
def _sc_kernels():
    mesh = plsc.VectorSubcoreMesh(core_axis_name="c", subcore_axis_name="s")
    params = pltpu.CompilerParams(use_tc_tiling_on_sc=False)
    sc_deg = functools.partial(
        pl.kernel,
        out_type=jax.ShapeDtypeStruct((NP, 16), jnp.float32),
        mesh=mesh,
        scratch_types=[
            pltpu.VMEM((K,), jnp.int32),
            pltpu.VMEM((K, 16), jnp.float32),
            pltpu.VMEM((ZST, 16), jnp.float32),
            pltpu.VMEM_SHARED((NH + KL, 16), jnp.float32),
        ],
        compiler_params=params,
    )(_sc_deg_body)
    sc_edge = functools.partial(
        pl.kernel,
        out_type=jax.ShapeDtypeStruct((NP, 16), jnp.float32),
        mesh=mesh,
        scratch_types=[
            pltpu.VMEM((K,), jnp.int32),
            pltpu.VMEM((K,), jnp.int32),
            pltpu.VMEM((K, 16), jnp.float32),
            pltpu.VMEM((ZST, 16), jnp.float32),
            pltpu.VMEM_SHARED((NH + KL, 16), jnp.float32),
            pltpu.SemaphoreType.DMA,
            pltpu.SemaphoreType.DMA,
        ],
        compiler_params=params,
    )(_sc_edge_body)
    return sc_deg, sc_edge


def _t1_body(deg_ref, state_ref, w_ref, dis_ref, hs_ref):
    deg = deg_ref[:, 0:1]
    dis = lax.rsqrt(deg + 1.0)
    dis_ref[...] = dis
    hs_ref[...] = jnp.dot(state_ref[...], w_ref[...],
                          preferred_element_type=jnp.float32) * dis


def _t1(deg, state_p, w1):
    return pl.pallas_call(
        _t1_body,
        grid=(NGRID,),
        in_specs=[
            pl.BlockSpec((NBLK, 16), lambda i: (i, 0)),
            pl.BlockSpec((NBLK, C), lambda i: (i, 0)),
            pl.BlockSpec((C, C), lambda i: (0, 0)),
        ],
        out_specs=[
            pl.BlockSpec((NBLK, 1), lambda i: (i, 0)),
            pl.BlockSpec((NBLK, C), lambda i: (i, 0)),
        ],
        out_shape=[
            jax.ShapeDtypeStruct((NP, 1), jnp.float32),
            jax.ShapeDtypeStruct((NP, C), jnp.float32),
        ],
    )(deg, state_p, w1)


def _tc_body(a0, a1, a2, a3, hs_ref, dis_ref, b_ref, w_ref, x_ref, hsn_ref):
    dis = dis_ref[...]
    accs = [a0, a1, a2, a3]
    for q in range(4):
        aq = accs[q][...] + hs_ref[:, q * 16:(q + 1) * 16]
        pre = dis * aq + b_ref[:, q * 16:(q + 1) * 16]
        x_ref[:, q * 16:(q + 1) * 16] = jnp.maximum(pre, 0.0)
    x = x_ref[...]
    hsn_ref[...] = jnp.dot(x, w_ref[...],
                           preferred_element_type=jnp.float32) * dis


def _tc_combine(accs, hs, dis, b_row, w_next):
    qspec = pl.BlockSpec((NBLK, 16), lambda i: (i, 0))
    return pl.pallas_call(
        _tc_body,
        grid=(NGRID,),
        in_specs=[
            qspec, qspec, qspec, qspec,
            pl.BlockSpec((NBLK, C), lambda i: (i, 0)),
            pl.BlockSpec((NBLK, 1), lambda i: (i, 0)),
            pl.BlockSpec((1, C), lambda i: (0, 0)),
            pl.BlockSpec((C, C), lambda i: (0, 0)),
        ],
        out_specs=[
            pl.BlockSpec((NBLK, C), lambda i: (i, 0)),
            pl.BlockSpec((NBLK, C), lambda i: (i, 0)),
        ],
        out_shape=[
            jax.ShapeDtypeStruct((NP, C), jnp.float32),
            jax.ShapeDtypeStruct((NP, C), jnp.float32),
        ],
    )(*accs, hs, dis, b_row, w_next)


def _tc_last_body(a0, a1, a2, a3, hs_ref, dis_ref, b_ref, x_ref):
    dis = dis_ref[...]
    accs = [a0, a1, a2, a3]
    for q in range(4):
        aq = accs[q][...] + hs_ref[:, q * 16:(q + 1) * 16]
        pre = dis * aq + b_ref[:, q * 16:(q + 1) * 16]
        x_ref[:, q * 16:(q + 1) * 16] = jnp.maximum(pre, 0.0)


def _tc_last(accs, hs, dis, b_row):
    qspec = pl.BlockSpec((NBLK, 16), lambda i: (i, 0))
    return pl.pallas_call(
        _tc_last_body,
        grid=(NGRID,),
        in_specs=[
            qspec, qspec, qspec, qspec,
            pl.BlockSpec((NBLK, C), lambda i: (i, 0)),
            pl.BlockSpec((NBLK, 1), lambda i: (i, 0)),
            pl.BlockSpec((1, C), lambda i: (0, 0)),
        ],
        out_specs=pl.BlockSpec((NBLK, C), lambda i: (i, 0)),
        out_shape=jax.ShapeDtypeStruct((NP, C), jnp.float32),
    )(*accs, hs, dis, b_row)


def _head_body(x1, x2, x3, x4, x5, st, w1s_ref, ws_ref, wt_ref, posb_ref,
               w2_ref, b2_ref, w3_ref, b3_ref, act_ref, reg_ref):
    f32 = jnp.float32
    y = posb_ref[...]
    xs = [x1, x2, x3, x4, x5]
    for l in range(5):
        y = y + jnp.dot(xs[l][...], w1s_ref[l], preferred_element_type=f32)
    y = y + jnp.dot(st[...], ws_ref[...], preferred_element_type=f32)
    nb = HBLK // ACT
    m_bn = (lax.broadcasted_iota(jnp.int32, (nb, HBLK), 1) // ACT ==
            lax.broadcasted_iota(jnp.int32, (nb, HBLK), 0)).astype(f32)
    m_nb = (lax.broadcasted_iota(jnp.int32, (HBLK, nb), 0) // ACT ==
            lax.broadcasted_iota(jnp.int32, (HBLK, nb), 1)).astype(f32)
    stc = st[:, 1:2]
    tot = jnp.dot(m_bn, stc, preferred_element_type=f32)
    tot_pn = jnp.dot(m_nb, tot, preferred_element_type=f32)
    y = y + tot_pn * wt_ref[...]
    y = jnp.where(y > 0, y, 0.01 * y)
    z = jnp.dot(y, w2_ref[...], preferred_element_type=f32) + b2_ref[...]
    z = jnp.where(z > 0, z, 0.01 * z)
    u = jnp.dot(z, w3_ref[...], preferred_element_type=f32) + b3_ref[...]
    conc = jnp.maximum(u, 0.0) + jnp.log(1.0 + jnp.exp(-jnp.abs(u)))
    den = jnp.dot(m_bn, conc, preferred_element_type=f32)
    den_pn = jnp.dot(m_nb, den, preferred_element_type=f32)
    act_ref[...] = conc / (den_pn + 1e-20)
    s = jnp.sum(jnp.abs(conc), keepdims=True)
    i = pl.program_id(0)

    @pl.when(i == 0)
    def _init():
        reg_ref[...] = s

    @pl.when(i != 0)
    def _acc():
        reg_ref[...] = reg_ref[...] + s


def _head(x1, x2, x3, x4, x5, state_p, w1_stack, ws, wt, posb, w2, b2, w3, b3):
    xspec = pl.BlockSpec((HBLK, C), lambda i: (i, 0))
    return pl.pallas_call(
        _head_body,
        grid=(HGRID,),
        in_specs=[
            xspec, xspec, xspec, xspec, xspec, xspec,
            pl.BlockSpec((5, C, 32), lambda i: (0, 0, 0)),
            pl.BlockSpec((C, 32), lambda i: (0, 0)),
            pl.BlockSpec((1, 32), lambda i: (0, 0)),
            pl.BlockSpec((HBLK, 32), lambda i: (0, 0)),
            pl.BlockSpec((32, 32), lambda i: (0, 0)),
            pl.BlockSpec((1, 32), lambda i: (0, 0)),
            pl.BlockSpec((32, 1), lambda i: (0, 0)),
            pl.BlockSpec((1, 1), lambda i: (0, 0)),
        ],
        out_specs=[
            pl.BlockSpec((HBLK, 1), lambda i: (i, 0)),
            pl.BlockSpec((1, 1), lambda i: (0, 0)),
        ],
        out_shape=[
            jax.ShapeDtypeStruct((N, 1), jnp.float32),
            jax.ShapeDtypeStruct((1, 1), jnp.float32),
        ],
    )(x1, x2, x3, x4, x5, state_p, w1_stack, ws, wt, posb, w2, b2, w3, b3)


def kernel(state, edge_index, W1, b1, W2, b2, W3, b3,
           lin1_W, lin1_b, lin2_W, lin2_b, lin3_W, lin3_b):
    i32 = jnp.int32
    f32 = jnp.float32

    row = edge_index[0]
    col = edge_index[1]
    row_p = jnp.concatenate([row, jnp.zeros((EP - E,), i32)])
    col_p = jnp.concatenate([col, jnp.full((EP - E,), N, i32)])
    rows4 = (row_p[None, :] * 4 + jnp.arange(4, dtype=i32)[:, None]).reshape(
        4, NSUB, CHUNKS, K)
    local0 = jnp.where((col_p >= 0) & (col_p < NH), col_p, NH)
    local1c = col_p - NH
    local1 = jnp.where((local1c >= 0) & (local1c < NH), local1c, NH)
    colsc = jnp.stack([local0, local1]).reshape(2, NSUB, CHUNKS, K)

    state_p = jnp.zeros((NP, C), f32).at[:N].set(state)

    sc_deg, sc_edge = _sc_kernels()

    def edge_pass(hs):
        hs_flat = hs.reshape(4 * NP, 16)
        return [sc_edge(hs_flat, rows4[q], colsc) for q in range(4)]

    deg = sc_deg(colsc)
    dis, hs1 = _t1(deg, state_p, W1)

    b1r = b1.reshape(1, C)
    b2r = b2.reshape(1, C)
    b3r = b3.reshape(1, C)

    x1, hs2 = _tc_combine(edge_pass(hs1), hs1, dis, b1r, W2)
    x2, hs3 = _tc_combine(edge_pass(hs2), hs2, dis, b2r, W3)
    x3, hs4 = _tc_combine(edge_pass(hs3), hs3, dis, b3r, W3)
    x4, hs5 = _tc_combine(edge_pass(hs4), hs4, dis, b3r, W3)
    x5 = _tc_last(edge_pass(hs5), hs5, dis, b3r)

    w1_stack = lin1_W[:5 * C].reshape(5, C, 32)
    ws = lin1_W[5 * C:6 * C]
    wt = lin1_W[6 * C].reshape(1, 32)
    pos = _positions()
    pos_lin = pos @ lin1_W[6 * C + 1:] + lin1_b[None, :]
    posb = jnp.tile(pos_lin, (HBLK // ACT, 1))
    b2h = lin2_b.reshape(1, 32)
    b3h = lin3_b.reshape(1, 1)

    act_col, reg = _head(x1, x2, x3, x4, x5, state_p, w1_stack, ws, wt,
                         posb, lin2_W, b2h, lin3_W, b3h)

    action = act_col.reshape(BATCH, ACT)
    regularize = reg[0, 0] / jnp.float32(N)
    return (action, regularize)


# speedup vs baseline: 3.2431x; 1.0654x over previous
import functools

import jax
import jax.numpy as jnp
from jax import lax
from jax.experimental import pallas as pl
from jax.experimental.pallas import tpu as pltpu
from jax.experimental.pallas import tpu_sc as plsc
import numpy as np

N = 79000
C = 64
E = 1264000
ACT = 79
BATCH = N // ACT

NP = 79872
NH = NP // 2
NSUB = 16
KL = 128
KC = 16
K = KL * KC
CHUNKS = 40
PER_TILE = CHUNKS * K
EP = PER_TILE * NSUB
ZST = (NH + KL) // NSUB
OST = NH // NSUB
NBLK = 768
NGRID = NP // NBLK
HBLK = 40 * ACT
HGRID = BATCH // 40

_POS_INDICES = [120, 124, 128, 132, 136, 140, 144, 148, 152, 237, 241, 245,
                249, 253, 257, 261, 265, 269, 354, 358, 362, 366, 370, 374,
                378, 382, 386, 471, 475, 479, 483, 487, 491, 495, 499, 503,
                588, 592, 596, 600, 604, 608, 612, 616, 620, 705, 709, 713,
                717, 721, 725, 729, 733, 737, 822, 826, 830, 834, 838, 842,
                846, 850, 854, 48, 53, 60, 67, 73, 157, 352, 388, 583, 586,
                817, 901, 906, 913, 920, 926]


def _positions():
    width, height = 39, 25
    pf = np.zeros((ACT, 2), dtype=np.float32)
    for i, p in enumerate(_POS_INDICES):
        pf[i, 0] = (p % width) / (width - 1)
        pf[i, 1] = (p // width) / (height - 1)
    return jnp.asarray(pf)


def _sc_deg_body(colsc, out, idx_s, ones_b, zbig, acc):
    cid = lax.axis_index("c")
    sid = lax.axis_index("s")

    @pl.loop(0, K)
    def _fill(i):
        ones_b[i, :] = jnp.full((16,), 1.0, jnp.float32)

    @pl.loop(0, ZST)
    def _fz(i):
        zbig[i, :] = jnp.zeros((16,), jnp.float32)

    pltpu.sync_copy(zbig, acc.at[pl.ds(sid * ZST, ZST), :])
    plsc.subcore_barrier()

    @pl.loop(0, CHUNKS)
    def _chunk(ch):
        pltpu.sync_copy(colsc.at[cid, sid, ch], idx_s)
        pltpu.sync_copy(ones_b, acc.at[idx_s], add=True)

    plsc.subcore_barrier()
    pltpu.sync_copy(acc.at[pl.ds(sid * OST, OST), :],
                    out.at[pl.ds(cid * NH + sid * OST, OST)])


def _sc_edge_body(hs_flat, rowsq, colsc, zeros, out, idx_g0, idx_g1, idx_s0,
                  idx_s1, rows0, rows1, acc, sem_g0, sem_g1, sem_s0, sem_s1):
    cid = lax.axis_index("c")
    sid = lax.axis_index("s")

    pltpu.sync_copy(zeros, acc.at[pl.ds(sid * ZST, ZST), :])
    plsc.subcore_barrier()

    pltpu.sync_copy(rowsq.at[sid, 0], idx_g0)
    pltpu.sync_copy(colsc.at[cid, sid, 0], idx_s0)
    pltpu.async_copy(hs_flat.at[idx_g0], rows0, sem_g0)

    @pl.loop(0, CHUNKS // 2)
    def _pair(i):
        ch1 = 2 * i + 1
        chn = lax.min(2 * i + 2, CHUNKS - 1)
        pltpu.sync_copy(rowsq.at[sid, ch1], idx_g1)
        pltpu.sync_copy(colsc.at[cid, sid, ch1], idx_s1)
        pltpu.make_async_copy(hs_flat.at[idx_g0], rows0, sem_g0).wait()
        pltpu.async_copy(hs_flat.at[idx_g1], rows1, sem_g1)
        pltpu.async_copy(rows0, acc.at[idx_s0], sem_s0, add=True)
        pltpu.make_async_copy(hs_flat.at[idx_g1], rows1, sem_g1).wait()
        pltpu.make_async_copy(rows0, acc.at[idx_s0], sem_s0).wait()
        pltpu.async_copy(rows1, acc.at[idx_s1], sem_s1, add=True)
        pltpu.sync_copy(rowsq.at[sid, chn], idx_g0)
        pltpu.sync_copy(colsc.at[cid, sid, chn], idx_s0)
        pltpu.async_copy(hs_flat.at[idx_g0], rows0, sem_g0)
        pltpu.make_async_copy(rows1, acc.at[idx_s1], sem_s1).wait()

    pltpu.make_async_copy(hs_flat.at[idx_g0], rows0, sem_g0).wait()
    plsc.subcore_barrier()
    pltpu.sync_copy(acc.at[pl.ds(sid * OST, OST), :],
                    out.at[pl.ds(cid * NH + sid * OST, OST)])


@functools.lru_cache(maxsize=1)
def _sc_kernels():
    mesh = plsc.VectorSubcoreMesh(core_axis_name="c", subcore_axis_name="s")
    params = pltpu.CompilerParams(use_tc_tiling_on_sc=False)
    sc_deg = functools.partial(
        pl.kernel,
        out_type=jax.ShapeDtypeStruct((NP, 16), jnp.float32),
        mesh=mesh,
        scratch_types=[
            pltpu.VMEM((K,), jnp.int32),
            pltpu.VMEM((K, 16), jnp.float32),
            pltpu.VMEM((ZST, 16), jnp.float32),
            pltpu.VMEM_SHARED((NH + KL, 16), jnp.float32),
        ],
        compiler_params=params,
    )(_sc_deg_body)
    sc_edge = functools.partial(
        pl.kernel,
        out_type=jax.ShapeDtypeStruct((NP, 16), jnp.float32),
        mesh=mesh,
        scratch_types=[
            pltpu.VMEM((K,), jnp.int32),
            pltpu.VMEM((K,), jnp.int32),
            pltpu.VMEM((K,), jnp.int32),
            pltpu.VMEM((K,), jnp.int32),
            pltpu.VMEM((K, 16), jnp.float32),
            pltpu.VMEM((K, 16), jnp.float32),
            pltpu.VMEM_SHARED((NH + KL, 16), jnp.float32),
            pltpu.SemaphoreType.DMA,
            pltpu.SemaphoreType.DMA,
            pltpu.SemaphoreType.DMA,
            pltpu.SemaphoreType.DMA,
        ],
        compiler_params=params,
    )(_sc_edge_body)
    return sc_deg, sc_edge


def _t1_body(deg_ref, state_ref, w_ref, dis_ref, hs_ref):
    deg = deg_ref[:, 0:1]
    dis = lax.rsqrt(deg + 1.0)
    dis_ref[...] = dis
    hs_ref[...] = jnp.dot(state_ref[...], w_ref[...],
                          preferred_element_type=jnp.float32) * dis


def _t1(deg, state_p, w1):
    return pl.pallas_call(
        _t1_body,
        grid=(NGRID,),
        in_specs=[
            pl.BlockSpec((NBLK, 16), lambda i: (i, 0)),
            pl.BlockSpec((NBLK, C), lambda i: (i, 0)),
            pl.BlockSpec((C, C), lambda i: (0, 0)),
        ],
        out_specs=[
            pl.BlockSpec((NBLK, 1), lambda i: (i, 0)),
            pl.BlockSpec((NBLK, C), lambda i: (i, 0)),
        ],
        out_shape=[
            jax.ShapeDtypeStruct((NP, 1), jnp.float32),
            jax.ShapeDtypeStruct((NP, C), jnp.float32),
        ],
    )(deg, state_p, w1)


def _tc_body(a0, a1, a2, a3, hs_ref, dis_ref, b_ref, w_ref, x_ref, hsn_ref):
    dis = dis_ref[...]
    accs = [a0, a1, a2, a3]
    for q in range(4):
        aq = accs[q][...] + hs_ref[:, q * 16:(q + 1) * 16]
        pre = dis * aq + b_ref[:, q * 16:(q + 1) * 16]
        x_ref[:, q * 16:(q + 1) * 16] = jnp.maximum(pre, 0.0)
    x = x_ref[...]
    hsn_ref[...] = jnp.dot(x, w_ref[...],
                           preferred_element_type=jnp.float32) * dis


def _tc_combine(accs, hs, dis, b_row, w_next):
    qspec = pl.BlockSpec((NBLK, 16), lambda i: (i, 0))
    return pl.pallas_call(
        _tc_body,
        grid=(NGRID,),
        in_specs=[
            qspec, qspec, qspec, qspec,
            pl.BlockSpec((NBLK, C), lambda i: (i, 0)),
            pl.BlockSpec((NBLK, 1), lambda i: (i, 0)),
            pl.BlockSpec((1, C), lambda i: (0, 0)),
            pl.BlockSpec((C, C), lambda i: (0, 0)),
        ],
        out_specs=[
            pl.BlockSpec((NBLK, C), lambda i: (i, 0)),
            pl.BlockSpec((NBLK, C), lambda i: (i, 0)),
        ],
        out_shape=[
            jax.ShapeDtypeStruct((NP, C), jnp.float32),
            jax.ShapeDtypeStruct((NP, C), jnp.float32),
        ],
    )(*accs, hs, dis, b_row, w_next)


def _tc_last_body(a0, a1, a2, a3, hs_ref, dis_ref, b_ref, x_ref):
    dis = dis_ref[...]
    accs = [a0, a1, a2, a3]
    for q in range(4):
        aq = accs[q][...] + hs_ref[:, q * 16:(q + 1) * 16]
        pre = dis * aq + b_ref[:, q * 16:(q + 1) * 16]
        x_ref[:, q * 16:(q + 1) * 16] = jnp.maximum(pre, 0.0)


def _tc_last(accs, hs, dis, b_row):
    qspec = pl.BlockSpec((NBLK, 16), lambda i: (i, 0))
    return pl.pallas_call(
        _tc_last_body,
        grid=(NGRID,),
        in_specs=[
            qspec, qspec, qspec, qspec,
            pl.BlockSpec((NBLK, C), lambda i: (i, 0)),
            pl.BlockSpec((NBLK, 1), lambda i: (i, 0)),
            pl.BlockSpec((1, C), lambda i: (0, 0)),
        ],
        out_specs=pl.BlockSpec((NBLK, C), lambda i: (i, 0)),
        out_shape=jax.ShapeDtypeStruct((NP, C), jnp.float32),
    )(*accs, hs, dis, b_row)


def _head_body(x1, x2, x3, x4, x5, st, w1s_ref, ws_ref, wt_ref, posb_ref,
               w2_ref, b2_ref, w3_ref, b3_ref, act_ref, reg_ref):
    f32 = jnp.float32
    y = posb_ref[...]
    xs = [x1, x2, x3, x4, x5]
    for l in range(5):
        y = y + jnp.dot(xs[l][...], w1s_ref[l], preferred_element_type=f32)
    y = y + jnp.dot(st[...], ws_ref[...], preferred_element_type=f32)
    nb = HBLK // ACT
    m_bn = (lax.broadcasted_iota(jnp.int32, (nb, HBLK), 1) // ACT ==
            lax.broadcasted_iota(jnp.int32, (nb, HBLK), 0)).astype(f32)
    m_nb = (lax.broadcasted_iota(jnp.int32, (HBLK, nb), 0) // ACT ==
            lax.broadcasted_iota(jnp.int32, (HBLK, nb), 1)).astype(f32)
    stc = st[:, 1:2]
    tot = jnp.dot(m_bn, stc, preferred_element_type=f32)
    tot_pn = jnp.dot(m_nb, tot, preferred_element_type=f32)
    y = y + tot_pn * wt_ref[...]
    y = jnp.where(y > 0, y, 0.01 * y)
    z = jnp.dot(y, w2_ref[...], preferred_element_type=f32) + b2_ref[...]
    z = jnp.where(z > 0, z, 0.01 * z)
    u = jnp.dot(z, w3_ref[...], preferred_element_type=f32) + b3_ref[...]
    conc = jnp.maximum(u, 0.0) + jnp.log(1.0 + jnp.exp(-jnp.abs(u)))
    den = jnp.dot(m_bn, conc, preferred_element_type=f32)
    den_pn = jnp.dot(m_nb, den, preferred_element_type=f32)
    act_ref[...] = conc / (den_pn + 1e-20)
    s = jnp.sum(jnp.abs(conc), keepdims=True)
    i = pl.program_id(0)

    @pl.when(i == 0)
    def _init():
        reg_ref[...] = s

    @pl.when(i != 0)
    def _acc():
        reg_ref[...] = reg_ref[...] + s


def _head(x1, x2, x3, x4, x5, state_p, w1_stack, ws, wt, posb, w2, b2, w3, b3):
    xspec = pl.BlockSpec((HBLK, C), lambda i: (i, 0))
    return pl.pallas_call(
        _head_body,
        grid=(HGRID,),
        in_specs=[
            xspec, xspec, xspec, xspec, xspec, xspec,
            pl.BlockSpec((5, C, 32), lambda i: (0, 0, 0)),
            pl.BlockSpec((C, 32), lambda i: (0, 0)),
            pl.BlockSpec((1, 32), lambda i: (0, 0)),
            pl.BlockSpec((HBLK, 32), lambda i: (0, 0)),
            pl.BlockSpec((32, 32), lambda i: (0, 0)),
            pl.BlockSpec((1, 32), lambda i: (0, 0)),
            pl.BlockSpec((32, 1), lambda i: (0, 0)),
            pl.BlockSpec((1, 1), lambda i: (0, 0)),
        ],
        out_specs=[
            pl.BlockSpec((HBLK, 1), lambda i: (i, 0)),
            pl.BlockSpec((1, 1), lambda i: (0, 0)),
        ],
        out_shape=[
            jax.ShapeDtypeStruct((N, 1), jnp.float32),
            jax.ShapeDtypeStruct((1, 1), jnp.float32),
        ],
    )(x1, x2, x3, x4, x5, state_p, w1_stack, ws, wt, posb, w2, b2, w3, b3)


def kernel(state, edge_index, W1, b1, W2, b2, W3, b3,
           lin1_W, lin1_b, lin2_W, lin2_b, lin3_W, lin3_b):
    i32 = jnp.int32
    f32 = jnp.float32

    row = edge_index[0]
    col = edge_index[1]
    row_p = jnp.concatenate([row, jnp.zeros((EP - E,), i32)])
    col_p = jnp.concatenate([col, jnp.full((EP - E,), N, i32)])
    rows4 = (row_p[None, :] * 4 + jnp.arange(4, dtype=i32)[:, None]).reshape(
        4, NSUB, CHUNKS, K)
    local0 = jnp.where((col_p >= 0) & (col_p < NH), col_p, NH)
    local1c = col_p - NH
    local1 = jnp.where((local1c >= 0) & (local1c < NH), local1c, NH)
    colsc = jnp.stack([local0, local1]).reshape(2, NSUB, CHUNKS, K)

    state_p = jnp.zeros((NP, C), f32).at[:N].set(state)

    sc_deg, sc_edge = _sc_kernels()

    zeros_st = jnp.zeros((ZST, 16), f32)

    def edge_pass(hs):
        hs_flat = hs.reshape(4 * NP, 16)
        return [sc_edge(hs_flat, rows4[q], colsc, zeros_st) for q in range(4)]

    deg = sc_deg(colsc)
    dis, hs1 = _t1(deg, state_p, W1)

    b1r = b1.reshape(1, C)
    b2r = b2.reshape(1, C)
    b3r = b3.reshape(1, C)

    x1, hs2 = _tc_combine(edge_pass(hs1), hs1, dis, b1r, W2)
    x2, hs3 = _tc_combine(edge_pass(hs2), hs2, dis, b2r, W3)
    x3, hs4 = _tc_combine(edge_pass(hs3), hs3, dis, b3r, W3)
    x4, hs5 = _tc_combine(edge_pass(hs4), hs4, dis, b3r, W3)
    x5 = _tc_last(edge_pass(hs5), hs5, dis, b3r)

    w1_stack = lin1_W[:5 * C].reshape(5, C, 32)
    ws = lin1_W[5 * C:6 * C]
    wt = lin1_W[6 * C].reshape(1, 32)
    pos = _positions()
    pos_lin = pos @ lin1_W[6 * C + 1:] + lin1_b[None, :]
    posb = jnp.tile(pos_lin, (HBLK // ACT, 1))
    b2h = lin2_b.reshape(1, 32)
    b3h = lin3_b.reshape(1, 1)

    act_col, reg = _head(x1, x2, x3, x4, x5, state_p, w1_stack, ws, wt,
                         posb, lin2_W, b2h, lin3_W, b3h)

    action = act_col.reshape(BATCH, ACT)
    regularize = reg[0, 0] / jnp.float32(N)
    return (action, regularize)
